# baseline (speedup 1.0000x reference)
import numpy as np
import jax
import jax.numpy as jnp
from jax.experimental import pallas as pl
from jax.experimental.pallas import tpu as pltpu

_B = 128
_V = 100000
_BV = 10000
_NS = _V // _BV
_CH = 200
_NCH = _BV // _CH

_KS0 = np.uint32(0)
_KS1 = np.uint32(42)
_KS2 = np.uint32(np.uint32(0x1BD11BDA) ^ np.uint32(42))
_TINY = np.float32(np.finfo(np.float32).tiny)
_INT_MAX = np.int32(2**31 - 1)


def _rotl(x, r):
    return jax.lax.shift_left(x, np.uint32(r)) | jax.lax.shift_right_logical(
        x, np.uint32(32 - r)
    )


def _threefry_xor_bits(lin):
    x0 = jnp.zeros_like(lin)
    x1 = lin
    ks = (_KS0, _KS1, _KS2)
    rots = ((13, 15, 26, 6), (17, 29, 16, 24))
    for i in range(5):
        for r in rots[i % 2]:
            x0 = x0 + x1
            x1 = _rotl(x1, r) ^ x0
        x0 = x0 + ks[(i + 1) % 3]
        x1 = x1 + ks[(i + 2) % 3] + np.uint32(i + 1)
    return x0 ^ x1


def _perturbed(p, lin):
    bits = _threefry_xor_bits(lin)
    fb = jax.lax.bitcast_convert_type(
        (bits >> np.uint32(9)) | np.uint32(0x3F800000), jnp.float32
    ) - np.float32(1.0)
    u = fb + _TINY
    return -jnp.log(-jnp.log(u)) + jnp.log(jnp.maximum(p, np.float32(1e-30)))


def _sample_kernel(p_ref, res_ref, vc_ref, m_ref, c_ref, q_ref):
    s = pl.program_id(0)

    @pl.when(s == 0)
    def _init():
        m_ref[...] = jnp.full((_CH, _B), -jnp.inf, jnp.float32)
        c_ref[...] = jnp.zeros((_CH, _B), jnp.int32)
        q_ref[...] = jnp.zeros((_CH, _B), jnp.float32)

    lane_u = jax.lax.broadcasted_iota(jnp.uint32, (_CH, _B), 1)
    row_u = jax.lax.broadcasted_iota(jnp.uint32, (_CH, _B), 0)
    lin0 = lane_u * np.uint32(_V) + (row_u + _KS1)
    base_u = (s * _BV).astype(jnp.uint32)
    base_i = s * _BV

    m = m_ref[...]
    c = c_ref[...]
    q = q_ref[...]
    for k in range(_NCH):
        off = k * _CH
        pch = p_ref[pl.ds(off, _CH), :]
        val = _perturbed(pch, lin0 + (base_u + np.uint32(off)))
        upd = val > m
        m = jnp.where(upd, val, m)
        c = jnp.where(upd, jnp.broadcast_to(base_i + np.int32(off), (_CH, _B)), c)
        q = jnp.where(upd, pch, q)
    m_ref[...] = m
    c_ref[...] = c
    q_ref[...] = q

    @pl.when(s == _NS - 1)
    def _fold():
        row_i = jax.lax.broadcasted_iota(jnp.int32, (_CH, _B), 0)
        cg = c + row_i
        bm = jnp.max(m, axis=0, keepdims=True)
        is_m = m == bm
        idxf = jnp.min(jnp.where(is_m, cg, _INT_MAX), axis=0, keepdims=True)
        qf = jnp.max(
            jnp.where(is_m & (cg == idxf), q, np.float32(-1.0)),
            axis=0, keepdims=True,
        )
        res_ref[...] = idxf
        vc_ref[...] = (np.float32(1.0) - qf) / jnp.maximum(qf, np.float32(1e-10))


def _variance_kernel(p_ref, idx_ref, vc_ref, out_ref):
    s = pl.program_id(0)
    base_i = s * _BV
    row_i = jax.lax.broadcasted_iota(jnp.int32, (_CH, _B), 0)
    idxf = idx_ref[...]
    vc = vc_ref[...]
    for k in range(_NCH):
        off = k * _CH
        pch = p_ref[pl.ds(off, _CH), :]
        idx_sh = idxf - (base_i + np.int32(off))
        vnc = pch / jnp.maximum(np.float32(1.0) - pch, np.float32(1e-10))
        out_ref[pl.ds(off, _CH), :] = jnp.where(row_i == idx_sh, vc, vnc)


def kernel(p, alpha, beta):
    del alpha, beta
    pt = p.T
    res_t, vc_t = pl.pallas_call(
        _sample_kernel,
        grid=(_NS,),
        in_specs=[pl.BlockSpec((_BV, _B), lambda s: (s, 0))],
        out_specs=[
            pl.BlockSpec((1, _B), lambda s: (0, 0)),
            pl.BlockSpec((1, _B), lambda s: (0, 0)),
        ],
        out_shape=[
            jax.ShapeDtypeStruct((1, _B), jnp.int32),
            jax.ShapeDtypeStruct((1, _B), jnp.float32),
        ],
        scratch_shapes=[
            pltpu.VMEM((_CH, _B), jnp.float32),
            pltpu.VMEM((_CH, _B), jnp.int32),
            pltpu.VMEM((_CH, _B), jnp.float32),
        ],
    )(pt)

    unc_t = pl.pallas_call(
        _variance_kernel,
        grid=(_NS,),
        in_specs=[
            pl.BlockSpec((_BV, _B), lambda s: (s, 0)),
            pl.BlockSpec((1, _B), lambda s: (0, 0)),
            pl.BlockSpec((1, _B), lambda s: (0, 0)),
        ],
        out_specs=pl.BlockSpec((_BV, _B), lambda s: (s, 0)),
        out_shape=jax.ShapeDtypeStruct((_V, _B), jnp.float32),
    )(pt, res_t, vc_t)

    return (res_t.T, unc_t.T)


# speedup vs baseline: 1.0130x; 1.0130x over previous
import numpy as np
import jax
import jax.numpy as jnp
from jax.experimental import pallas as pl
from jax.experimental.pallas import tpu as pltpu

_B = 128
_V = 100000
_BV = 10000
_NS = _V // _BV
_CH = 200
_NCH = _BV // _CH

_KS0 = np.uint32(0)
_KS1 = np.uint32(42)
_KS2 = np.uint32(np.uint32(0x1BD11BDA) ^ np.uint32(42))
_TINY = np.float32(np.finfo(np.float32).tiny)
_SCALE = np.float32(1.0 - np.finfo(np.float32).tiny)
_INT_MAX = np.int32(2**31 - 1)


def _rotl(x, r):
    return jax.lax.shift_left(x, np.uint32(r)) | jax.lax.shift_right_logical(
        x, np.uint32(32 - r)
    )


def _threefry_xor_bits(lin):
    x0 = jnp.zeros_like(lin)
    x1 = lin
    ks = (_KS0, _KS1, _KS2)
    rots = ((13, 15, 26, 6), (17, 29, 16, 24))
    for i in range(5):
        for r in rots[i % 2]:
            x0 = x0 + x1
            x1 = _rotl(x1, r) ^ x0
        x0 = x0 + ks[(i + 1) % 3]
        x1 = x1 + ks[(i + 2) % 3] + np.uint32(i + 1)
    return x0 ^ x1


def _perturbed(p, lin):
    bits = _threefry_xor_bits(lin)
    fb = jax.lax.bitcast_convert_type(
        (bits >> np.uint32(9)) | np.uint32(0x3F800000), jnp.float32
    ) - np.float32(1.0)
    u = fb + _TINY
    return -jnp.log(-jnp.log(u)) + jnp.log(jnp.maximum(p, np.float32(1e-30)))


def _main_kernel(p_ref, out_ref, res_ref, m_ref, c_ref, q_ref, if_ref, vc_ref):
    ph = pl.program_id(0)
    s = pl.program_id(1)

    @pl.when((ph == 0) & (s == 0))
    def _init():
        m_ref[...] = jnp.full((_CH, _B), -jnp.inf, jnp.float32)
        c_ref[...] = jnp.zeros((_CH, _B), jnp.int32)
        q_ref[...] = jnp.zeros((_CH, _B), jnp.float32)

    lane_u = jax.lax.broadcasted_iota(jnp.uint32, (_CH, _B), 1)
    row_u = jax.lax.broadcasted_iota(jnp.uint32, (_CH, _B), 0)
    row_i = jax.lax.broadcasted_iota(jnp.int32, (_CH, _B), 0)
    lin0 = lane_u * np.uint32(_V) + (row_u + _KS1)
    base_u = (s * _BV).astype(jnp.uint32)
    base_i = s * _BV

    @pl.when(ph == 0)
    def _phase0():
        m = m_ref[...]
        c = c_ref[...]
        q = q_ref[...]
        for k in range(_NCH):
            off = k * _CH
            pch = p_ref[pl.ds(off, _CH), :]
            val = _perturbed(pch, lin0 + (base_u + np.uint32(off)))
            upd = val > m
            m = jnp.where(upd, val, m)
            c = jnp.where(upd, jnp.broadcast_to(base_i + np.int32(off), (_CH, _B)), c)
            q = jnp.where(upd, pch, q)
        m_ref[...] = m
        c_ref[...] = c
        q_ref[...] = q

    @pl.when((ph == 1) & (s == 0))
    def _fold():
        m = m_ref[...]
        c = c_ref[...] + row_i
        q = q_ref[...]
        bm = jnp.max(m, axis=0, keepdims=True)
        is_m = m == bm
        idxf = jnp.min(jnp.where(is_m, c, _INT_MAX), axis=0, keepdims=True)
        qf = jnp.max(
            jnp.where(is_m & (c == idxf), q, np.float32(-1.0)),
            axis=0, keepdims=True,
        )
        if_ref[...] = idxf
        vc_ref[...] = (np.float32(1.0) - qf) / jnp.maximum(qf, np.float32(1e-10))
        res_ref[...] = idxf

    @pl.when(ph == 1)
    def _phase1():
        idxf = if_ref[...]
        vc = vc_ref[...]
        for k in range(_NCH):
            off = k * _CH
            pch = p_ref[pl.ds(off, _CH), :]
            idx_sh = idxf - (base_i + np.int32(off))
            vnc = pch / jnp.maximum(np.float32(1.0) - pch, np.float32(1e-10))
            out_ref[pl.ds(off, _CH), :] = jnp.where(row_i == idx_sh, vc, vnc)


def kernel(p, alpha, beta):
    del alpha, beta
    pt = p.T
    unc_t, res_t = pl.pallas_call(
        _main_kernel,
        grid=(2, _NS),
        in_specs=[pl.BlockSpec((_BV, _B), lambda ph, s: (s, 0))],
        out_specs=[
            pl.BlockSpec((_BV, _B), lambda ph, s: (jnp.where(ph == 1, s, 0), 0)),
            pl.BlockSpec((1, _B), lambda ph, s: (0, 0)),
        ],
        out_shape=[
            jax.ShapeDtypeStruct((_V, _B), jnp.float32),
            jax.ShapeDtypeStruct((1, _B), jnp.int32),
        ],
        scratch_shapes=[
            pltpu.VMEM((_CH, _B), jnp.float32),
            pltpu.VMEM((_CH, _B), jnp.int32),
            pltpu.VMEM((_CH, _B), jnp.float32),
            pltpu.VMEM((1, _B), jnp.int32),
            pltpu.VMEM((1, _B), jnp.float32),
        ],
    )(pt)
    return (res_t.T, unc_t.T)


# speedup vs baseline: 1.0922x; 1.0781x over previous
import numpy as np
import jax
import jax.numpy as jnp
from jax.experimental import pallas as pl
from jax.experimental.pallas import tpu as pltpu

_B = 128
_V = 100000
_BV = 10000
_NS = _V // _BV
_CH = 200
_NCH = _BV // _CH

_KS0 = np.uint32(0)
_KS1 = np.uint32(42)
_KS2 = np.uint32(np.uint32(0x1BD11BDA) ^ np.uint32(42))
_TINY = np.float32(np.finfo(np.float32).tiny)
_INT_MAX = np.int32(2**31 - 1)


def _rotl(x, r):
    return jax.lax.shift_left(x, np.uint32(r)) | jax.lax.shift_right_logical(
        x, np.uint32(32 - r)
    )


def _threefry_xor_bits(lin):
    x0 = jnp.zeros_like(lin)
    x1 = lin
    ks = (_KS0, _KS1, _KS2)
    rots = ((13, 15, 26, 6), (17, 29, 16, 24))
    for i in range(5):
        for r in rots[i % 2]:
            x0 = x0 + x1
            x1 = _rotl(x1, r) ^ x0
        x0 = x0 + ks[(i + 1) % 3]
        x1 = x1 + ks[(i + 2) % 3] + np.uint32(i + 1)
    return x0 ^ x1


def _perturbed(p, lin):
    bits = _threefry_xor_bits(lin)
    fb = jax.lax.bitcast_convert_type(
        (bits >> np.uint32(9)) | np.uint32(0x3F800000), jnp.float32
    ) - np.float32(1.0)
    u = fb + _TINY
    return -jnp.log(-jnp.log(u)) + jnp.log(jnp.maximum(p, np.float32(1e-30)))


def _main_kernel(p_ref, out_ref, res_ref, vcv_ref, m_ref, c_ref, q_ref):
    s = pl.program_id(0)

    @pl.when(s == 0)
    def _init():
        m_ref[...] = jnp.full((_CH, _B), -jnp.inf, jnp.float32)
        c_ref[...] = jnp.zeros((_CH, _B), jnp.int32)
        q_ref[...] = jnp.zeros((_CH, _B), jnp.float32)

    lane_u = jax.lax.broadcasted_iota(jnp.uint32, (_CH, _B), 1)
    row_u = jax.lax.broadcasted_iota(jnp.uint32, (_CH, _B), 0)
    lin0 = lane_u * np.uint32(_V) + (row_u + _KS1)
    base_u = (s * _BV).astype(jnp.uint32)
    base_i = s * _BV

    m = m_ref[...]
    c = c_ref[...]
    q = q_ref[...]
    for k in range(_NCH):
        off = k * _CH
        pch = p_ref[pl.ds(off, _CH), :]
        val = _perturbed(pch, lin0 + (base_u + np.uint32(off)))
        upd = val > m
        m = jnp.where(upd, val, m)
        c = jnp.where(upd, jnp.broadcast_to(base_i + np.int32(off), (_CH, _B)), c)
        q = jnp.where(upd, pch, q)
        out_ref[pl.ds(off, _CH), :] = pch / jnp.maximum(
            np.float32(1.0) - pch, np.float32(1e-10)
        )
    m_ref[...] = m
    c_ref[...] = c
    q_ref[...] = q

    @pl.when(s == _NS - 1)
    def _fold():
        row_i = jax.lax.broadcasted_iota(jnp.int32, (_CH, _B), 0)
        cg = c + row_i
        bm = jnp.max(m, axis=0, keepdims=True)
        is_m = m == bm
        idxf = jnp.min(jnp.where(is_m, cg, _INT_MAX), axis=0, keepdims=True)
        qf = jnp.max(
            jnp.where(is_m & (cg == idxf), q, np.float32(-1.0)),
            axis=0, keepdims=True,
        )
        res_ref[...] = idxf
        vcv_ref[...] = (np.float32(1.0) - qf) / jnp.maximum(qf, np.float32(1e-10))


def _patch_kernel(idx_sref, idxv_ref, vc_ref, u_in_ref, u_out_ref, rows_ref, sem):
    gathers = [
        pltpu.make_async_copy(
            u_in_ref.at[pl.ds(idx_sref[r], 1), :],
            rows_ref.at[pl.ds(r, 1), :],
            sem,
        )
        for r in range(_B)
    ]
    for cp in gathers:
        cp.start()
    for cp in gathers:
        cp.wait()
    idxv = idxv_ref[...]
    mask = jnp.broadcast_to(idxv, (_B, _B)) == jnp.reshape(idxv, (_B, 1))
    rows_ref[...] = jnp.where(
        mask, jnp.broadcast_to(vc_ref[...], (_B, _B)), rows_ref[...]
    )
    scatters = [
        pltpu.make_async_copy(
            rows_ref.at[pl.ds(r, 1), :],
            u_out_ref.at[pl.ds(idx_sref[r], 1), :],
            sem,
        )
        for r in range(_B)
    ]
    for cp in scatters:
        cp.start()
    for cp in scatters:
        cp.wait()


def kernel(p, alpha, beta):
    del alpha, beta
    pt = p.T
    unc_t, res_t, vc_t = pl.pallas_call(
        _main_kernel,
        grid=(_NS,),
        in_specs=[pl.BlockSpec((_BV, _B), lambda s: (s, 0))],
        out_specs=[
            pl.BlockSpec((_BV, _B), lambda s: (s, 0)),
            pl.BlockSpec((1, _B), lambda s: (0, 0)),
            pl.BlockSpec((1, _B), lambda s: (0, 0)),
        ],
        out_shape=[
            jax.ShapeDtypeStruct((_V, _B), jnp.float32),
            jax.ShapeDtypeStruct((1, _B), jnp.int32),
            jax.ShapeDtypeStruct((1, _B), jnp.float32),
        ],
        scratch_shapes=[
            pltpu.VMEM((_CH, _B), jnp.float32),
            pltpu.VMEM((_CH, _B), jnp.int32),
            pltpu.VMEM((_CH, _B), jnp.float32),
        ],
    )(pt)

    unc_t = pl.pallas_call(
        _patch_kernel,
        grid_spec=pltpu.PrefetchScalarGridSpec(
            num_scalar_prefetch=1,
            grid=(1,),
            in_specs=[
                pl.BlockSpec((1, _B), lambda i, idx: (0, 0)),
                pl.BlockSpec((1, _B), lambda i, idx: (0, 0)),
                pl.BlockSpec(memory_space=pl.ANY),
            ],
            out_specs=pl.BlockSpec(memory_space=pl.ANY),
            scratch_shapes=[
                pltpu.VMEM((_B, _B), jnp.float32),
                pltpu.SemaphoreType.DMA,
            ],
        ),
        out_shape=jax.ShapeDtypeStruct((_V, _B), jnp.float32),
        input_output_aliases={3: 0},
    )(res_t[0], res_t, vc_t, unc_t)

    return (res_t.T, unc_t.T)
